# Initial kernel scaffold; baseline (speedup 1.0000x reference)
import jax
import jax.numpy as jnp
from jax.experimental import pallas as pl


def kernel(module_logits, task_ids):
    raise NotImplementedError("write your pallas kernel here")


# speedup vs baseline: 1.7253x; 1.7253x over previous
import functools

import jax
import jax.numpy as jnp
from jax import lax
from jax.experimental import pallas as pl
from jax.experimental.pallas import tpu as pltpu
from jax.experimental.pallas import tpu_sc as plsc

N_TASKS = 1024
N_SPLITS = 8
N_SKILLS = 64
D = N_SPLITS * N_SKILLS
B = 16384
EPS = 1e-12

_NC = 2
_NS = 16
_NW = _NC * _NS

_B_PER_W = B // _NW
_CH = 128
_NCH = _B_PER_W // _CH


def _norm_body(table_ref, out_ref):
    x = table_ref[...]
    s = 1.0 / (1.0 + jnp.exp(-x))
    for g in range(N_SPLITS):
        sl = slice(g * N_SKILLS, (g + 1) * N_SKILLS)
        grp = s[:, sl]
        tot = jnp.sum(grp, axis=1, keepdims=True) + EPS
        out_ref[:, sl] = grp / tot


_normalize = pl.pallas_call(
    _norm_body,
    out_shape=jax.ShapeDtypeStruct((N_TASKS, D), jnp.float32),
)


def _gather_body(norm_hbm, ids_hbm, out_hbm, idx_v, rows_v, sem):
    wid = lax.axis_index("s") * _NC + lax.axis_index("c")
    base = wid * _B_PER_W
    for ch in range(_NCH):
        off = base + ch * _CH
        pltpu.sync_copy(ids_hbm.at[pl.ds(off, _CH)], idx_v)
        pltpu.async_copy(norm_hbm.at[idx_v], rows_v, sem).wait()
        pltpu.sync_copy(rows_v, out_hbm.at[pl.ds(off, _CH)])


_mesh = plsc.VectorSubcoreMesh(core_axis_name="c", subcore_axis_name="s")

_gather = functools.partial(
    pl.kernel,
    mesh=_mesh,
    out_type=jax.ShapeDtypeStruct((B, D), jnp.float32),
    scratch_types=[
        pltpu.VMEM((_CH,), jnp.int32),
        pltpu.VMEM((_CH, D), jnp.float32),
        pltpu.SemaphoreType.DMA,
    ],
)(_gather_body)


@jax.jit
def kernel(module_logits, task_ids):
    norm = _normalize(module_logits)
    out = _gather(norm, task_ids.astype(jnp.int32))
    return out.reshape(-1, N_SPLITS, N_SKILLS)
